# Initial kernel scaffold; baseline (speedup 1.0000x reference)
import jax
import jax.numpy as jnp
from jax.experimental import pallas as pl


def kernel(x_user, x_item, edge_item_user, edge_user_item, W_proj, b_proj, W_out, b_out, attn_l_iu, attn_r_iu, attn_l_ui, attn_r_ui):
    raise NotImplementedError("write your pallas kernel here")


# speedup vs baseline: 7.3061x; 7.3061x over previous
import functools

import jax
import jax.numpy as jnp
from jax import lax
from jax.experimental import pallas as pl
from jax.experimental.pallas import tpu as pltpu
from jax.experimental.pallas import tpu_sc as plsc

N = 10000
D = 128
E = 160000

NC = 2
NS = 16
KU = 80
UNITS_PER_TILE = E // (NS * KU)
IDX_BLOCKS = 25
UNITS_PER_BLOCK = UNITS_PER_TILE // IDX_BLOCKS
NP = 10240
ROWS_PER_TILE = NP // NS
DEGW = 16


def _proj_body(x_ref, wt_ref, b_ref, o_ref):
    o_ref[...] = (
        jnp.dot(x_ref[...], wt_ref[...], preferred_element_type=jnp.float32)
        + b_ref[...]
    )


def _project(x, wt, b2):
    bn = 2000
    return pl.pallas_call(
        _proj_body,
        grid=(x.shape[0] // bn,),
        in_specs=[
            pl.BlockSpec((bn, D), lambda i: (i, 0)),
            pl.BlockSpec((D, D), lambda i: (0, 0)),
            pl.BlockSpec((1, D), lambda i: (0, 0)),
        ],
        out_specs=pl.BlockSpec((bn, D), lambda i: (i, 0)),
        out_shape=jax.ShapeDtypeStruct((x.shape[0], D), jnp.float32),
    )(x, wt, b2)


def _sc_body(h_item, h_user, src_iu, dst_iu, src_ui, dst_ui,
             z80, ones_hbm, row_iota,
             agg_u, deg_u, agg_i, deg_i,
             sidx_v, didx_v, rows_a, rows_b, riota_v,
             idxu_a, idxu_b, idxu_c, sem, semb, semc,
             acc_sh):
    cid = lax.axis_index("c")
    sid = lax.axis_index("s")
    row0 = sid * ROWS_PER_TILE
    nblk = ROWS_PER_TILE // KU

    pltpu.sync_copy(z80, rows_a)
    pltpu.sync_copy(row_iota.at[sid], riota_v)

    def _stage_idxu(buf, src_ref, j):
        for _i in range(KU // 16):
            buf[pl.ds(_i * 16, 16)] = src_ref[j, pl.ds(_i * 16, 16)]

    def _zero_own_range():
        for _k in range(nblk):
            _stage_idxu(idxu_a, riota_v, _k)
            pltpu.async_copy(rows_a, acc_sh.at[idxu_a], sem).wait()

    def _copy_out(out_ref):
        def _out(k, c):
            sl = pl.ds(row0 + k * KU, KU)
            pltpu.async_copy(acc_sh.at[riota_v.at[k]], rows_a, sem).wait()
            pltpu.sync_copy(rows_a, out_ref.at[sl])
            return c

        lax.fori_loop(0, nblk, _out, 0)

    _zero_own_range()
    plsc.subcore_barrier()

    def _relation(table, src_idx, dst_idx, agg_out, deg_out):
        def _block1(b, carry):
            pltpu.sync_copy(src_idx.at[sid, b], sidx_v)
            pltpu.sync_copy(dst_idx.at[sid, b], didx_v)
            bufs = (rows_a, rows_b)
            sems = (sem, semb)
            pend = [
                pltpu.async_copy(table.at[sidx_v.at[0]], rows_a, sem),
                pltpu.async_copy(table.at[sidx_v.at[1]], rows_b, semb),
            ]
            for j in range(UNITS_PER_BLOCK):
                _stage_idxu(idxu_a, didx_v, j)
                pend[j % 2].wait()
                pltpu.sync_copy(bufs[j % 2], acc_sh.at[idxu_a], add=True)
                nxt = j + 2
                if nxt < UNITS_PER_BLOCK:
                    pend[nxt % 2] = pltpu.async_copy(
                        table.at[sidx_v.at[nxt]], bufs[nxt % 2], sems[nxt % 2])
            return carry

        lax.fori_loop(0, IDX_BLOCKS, _block1, 0)
        plsc.subcore_barrier()
        _copy_out(agg_out)

        pltpu.sync_copy(z80, rows_a)
        _zero_own_range()
        pltpu.sync_copy(ones_hbm, rows_b)
        plsc.subcore_barrier()

        def _block2(b, carry):
            pltpu.sync_copy(dst_idx.at[sid, b], didx_v)
            _stage_idxu(idxu_a, didx_v, 0)
            f0 = pltpu.async_copy(rows_b, acc_sh.at[idxu_a], sem, add=True)
            _stage_idxu(idxu_b, didx_v, 1)
            f1 = pltpu.async_copy(rows_b, acc_sh.at[idxu_b], semb, add=True)
            _stage_idxu(idxu_c, didx_v, 2)
            f2 = pltpu.async_copy(rows_b, acc_sh.at[idxu_c], semc, add=True)
            f0.wait()
            _stage_idxu(idxu_a, didx_v, 3)
            f3 = pltpu.async_copy(rows_b, acc_sh.at[idxu_a], sem, add=True)
            f1.wait()
            _stage_idxu(idxu_b, didx_v, 4)
            f4 = pltpu.async_copy(rows_b, acc_sh.at[idxu_b], semb, add=True)
            f2.wait()
            f3.wait()
            f4.wait()
            return carry

        lax.fori_loop(0, IDX_BLOCKS, _block2, 0)
        plsc.subcore_barrier()
        _copy_out(deg_out)

    @pl.when(cid == 0)
    def _():
        _relation(h_item, src_iu, dst_iu, agg_u, deg_u)

    @pl.when(cid == 1)
    def _():
        _relation(h_user, src_ui, dst_ui, agg_i, deg_i)


def _messages(h_user, h_item, src_iu, dst_iu, src_ui, dst_ui):
    mesh = plsc.VectorSubcoreMesh(core_axis_name="c", subcore_axis_name="s")
    z80 = jnp.zeros((KU, D), jnp.float32)
    ones = jnp.ones((KU, D), jnp.float32)
    row_iota = jnp.arange(NP, dtype=jnp.int32).reshape(
        NS, ROWS_PER_TILE // KU, KU)
    kern = functools.partial(
        pl.kernel,
        out_type=(
            jax.ShapeDtypeStruct((NP, D), jnp.float32),
            jax.ShapeDtypeStruct((NP, D), jnp.float32),
            jax.ShapeDtypeStruct((NP, D), jnp.float32),
            jax.ShapeDtypeStruct((NP, D), jnp.float32),
        ),
        mesh=mesh,
        scratch_types=[
            pltpu.VMEM((UNITS_PER_BLOCK, KU), jnp.int32),
            pltpu.VMEM((UNITS_PER_BLOCK, KU), jnp.int32),
            pltpu.VMEM((KU, D), jnp.float32),
            pltpu.VMEM((KU, D), jnp.float32),
            pltpu.VMEM((ROWS_PER_TILE // KU, KU), jnp.int32),
            pltpu.VMEM((KU,), jnp.int32),
            pltpu.VMEM((KU,), jnp.int32),
            pltpu.VMEM((KU,), jnp.int32),
            pltpu.SemaphoreType.DMA,
            pltpu.SemaphoreType.DMA,
            pltpu.SemaphoreType.DMA,
            pltpu.VMEM_SHARED((NP, D), jnp.float32),
        ],
    )(_sc_body)
    return kern(h_item, h_user, src_iu, dst_iu, src_ui, dst_ui,
                z80, ones, row_iota)


def _epi_body(scal_ref, agg_u, deg_u, h_u, agg_i, deg_i, h_i, wt, bo,
              out_u, out_i):
    al_iu = scal_ref[0, 0]
    ar_iu = scal_ref[0, 1]
    al_ui = scal_ref[0, 2]
    ar_ui = scal_ref[0, 3]
    pre_u = agg_u[...] * al_iu + h_u[...] * (deg_u[:, 0:1] * ar_iu)
    pre_u = jnp.where(pre_u > 0, pre_u, jnp.exp(pre_u) - 1.0)
    out_u[...] = (
        jnp.dot(pre_u, wt[...], preferred_element_type=jnp.float32) + bo[...]
    )
    pre_i = agg_i[...] * al_ui + h_i[...] * (deg_i[:, 0:1] * ar_ui)
    out_i[...] = jnp.where(pre_i > 0, pre_i, jnp.exp(pre_i) - 1.0)


def _epilogue(scal, agg_u, deg_u, h_u, agg_i, deg_i, h_i, wt, bo2):
    bn = 2000
    return pl.pallas_call(
        _epi_body,
        grid=(N // bn,),
        in_specs=[
            pl.BlockSpec(memory_space=pltpu.SMEM),
            pl.BlockSpec((bn, D), lambda i: (i, 0)),
            pl.BlockSpec((bn, D), lambda i: (i, 0)),
            pl.BlockSpec((bn, D), lambda i: (i, 0)),
            pl.BlockSpec((bn, D), lambda i: (i, 0)),
            pl.BlockSpec((bn, D), lambda i: (i, 0)),
            pl.BlockSpec((bn, D), lambda i: (i, 0)),
            pl.BlockSpec((D, D), lambda i: (0, 0)),
            pl.BlockSpec((1, D), lambda i: (0, 0)),
        ],
        out_specs=[
            pl.BlockSpec((bn, D), lambda i: (i, 0)),
            pl.BlockSpec((bn, D), lambda i: (i, 0)),
        ],
        out_shape=[
            jax.ShapeDtypeStruct((N, D), jnp.float32),
            jax.ShapeDtypeStruct((N, D), jnp.float32),
        ],
    )(scal, agg_u, deg_u, h_u, agg_i, deg_i, h_i, wt, bo2)


def kernel(x_user, x_item, edge_item_user, edge_user_item,
           W_proj, b_proj, W_out, b_out,
           attn_l_iu, attn_r_iu, attn_l_ui, attn_r_ui):
    wpt = W_proj.T
    wot = W_out.T
    bp2 = b_proj.reshape(1, D)
    bo2 = b_out.reshape(1, D)

    h_user = _project(x_user, wpt, bp2)
    h_item = _project(x_item, wpt, bp2)

    def _pack(edge):
        e = edge.astype(jnp.int32)
        shp = (NS, IDX_BLOCKS, UNITS_PER_BLOCK, KU)
        return e[0].reshape(shp), e[1].reshape(shp)

    src_iu, dst_iu = _pack(edge_item_user)
    src_ui, dst_ui = _pack(edge_user_item)

    agg_u, deg_u, agg_i, deg_i = _messages(
        h_user, h_item, src_iu, dst_iu, src_ui, dst_ui)

    scal = jnp.stack(
        [attn_l_iu[0], attn_r_iu[0], attn_l_ui[0], attn_r_ui[0]]
    ).reshape(1, 4)
    out_user, out_item = _epilogue(
        scal, agg_u, deg_u, h_user, agg_i, deg_i, h_item, wot, bo2)
    return out_user, out_item


# speedup vs baseline: 8.6374x; 1.1822x over previous
import functools

import jax
import jax.numpy as jnp
from jax import lax
from jax.experimental import pallas as pl
from jax.experimental.pallas import tpu as pltpu
from jax.experimental.pallas import tpu_sc as plsc

N = 10000
D = 128
E = 160000

NC = 2
NS = 16
KU = 80
UNITS_PER_TILE = E // (NS * KU)
IDX_BLOCKS = 5
UNITS_PER_BLOCK = UNITS_PER_TILE // IDX_BLOCKS
NP = 10240
ROWS_PER_TILE = NP // NS
DEGW = 16


def _proj_body(x_ref, wt_ref, b_ref, o_ref):
    o_ref[...] = (
        jnp.dot(x_ref[...], wt_ref[...], preferred_element_type=jnp.float32)
        + b_ref[...]
    )


def _project(x, wt, b2):
    bn = 2000
    return pl.pallas_call(
        _proj_body,
        grid=(x.shape[0] // bn,),
        in_specs=[
            pl.BlockSpec((bn, D), lambda i: (i, 0)),
            pl.BlockSpec((D, D), lambda i: (0, 0)),
            pl.BlockSpec((1, D), lambda i: (0, 0)),
        ],
        out_specs=pl.BlockSpec((bn, D), lambda i: (i, 0)),
        out_shape=jax.ShapeDtypeStruct((x.shape[0], D), jnp.float32),
    )(x, wt, b2)


def _sc_body(h_item, h_user, src_iu, dst_iu, src_ui, dst_ui,
             z80, ones_hbm, row_iota,
             agg_u, deg_u, agg_i, deg_i,
             sidx_v, didx_v, rows_a, rows_b, riota_v,
             idxu_a, idxu_b, idxu_c, sem, semb, semc,
             acc_sh):
    cid = lax.axis_index("c")
    sid = lax.axis_index("s")
    row0 = sid * ROWS_PER_TILE
    nblk = ROWS_PER_TILE // KU

    pltpu.sync_copy(z80, rows_a)
    pltpu.sync_copy(row_iota.at[sid], riota_v)

    def _stage_idxu(buf, src_ref, j):
        for _i in range(KU // 16):
            buf[pl.ds(_i * 16, 16)] = src_ref[j, pl.ds(_i * 16, 16)]

    def _zero_own_range():
        for _k in range(nblk):
            _stage_idxu(idxu_a, riota_v, _k)
            pltpu.async_copy(rows_a, acc_sh.at[idxu_a], sem).wait()

    def _copy_out(out_ref):
        def _out(k, c):
            sl = pl.ds(row0 + k * KU, KU)
            pltpu.async_copy(acc_sh.at[riota_v.at[k]], rows_a, sem).wait()
            pltpu.sync_copy(rows_a, out_ref.at[sl])
            return c

        lax.fori_loop(0, nblk, _out, 0)

    _zero_own_range()
    plsc.subcore_barrier()

    def _relation(table, src_idx, dst_idx, agg_out, deg_out):
        def _block1(b, carry):
            pltpu.sync_copy(src_idx.at[sid, b], sidx_v)
            pltpu.sync_copy(dst_idx.at[sid, b], didx_v)
            bufs = (rows_a, rows_b)
            sems = (sem, semb)
            pend = [
                pltpu.async_copy(table.at[sidx_v.at[0]], rows_a, sem),
                pltpu.async_copy(table.at[sidx_v.at[1]], rows_b, semb),
            ]
            for j in range(UNITS_PER_BLOCK):
                _stage_idxu(idxu_a, didx_v, j)
                pend[j % 2].wait()
                pltpu.sync_copy(bufs[j % 2], acc_sh.at[idxu_a], add=True)
                nxt = j + 2
                if nxt < UNITS_PER_BLOCK:
                    pend[nxt % 2] = pltpu.async_copy(
                        table.at[sidx_v.at[nxt]], bufs[nxt % 2], sems[nxt % 2])
            return carry

        lax.fori_loop(0, IDX_BLOCKS, _block1, 0)
        plsc.subcore_barrier()
        _copy_out(agg_out)

        pltpu.sync_copy(z80, rows_a)
        _zero_own_range()
        pltpu.sync_copy(ones_hbm, rows_b)
        plsc.subcore_barrier()

        def _block2(b, carry):
            pltpu.sync_copy(dst_idx.at[sid, b], didx_v)
            ibufs = (idxu_a, idxu_b, idxu_c)
            isems = (sem, semb, semc)
            pend = [None, None, None]
            for j in range(UNITS_PER_BLOCK):
                if pend[j % 3] is not None:
                    pend[j % 3].wait()
                _stage_idxu(ibufs[j % 3], didx_v, j)
                pend[j % 3] = pltpu.async_copy(
                    rows_b, acc_sh.at[ibufs[j % 3]], isems[j % 3], add=True)
            for p in pend:
                if p is not None:
                    p.wait()
            return carry

        lax.fori_loop(0, IDX_BLOCKS, _block2, 0)
        plsc.subcore_barrier()
        _copy_out(deg_out)

    @pl.when(cid == 0)
    def _():
        _relation(h_item, src_iu, dst_iu, agg_u, deg_u)

    @pl.when(cid == 1)
    def _():
        _relation(h_user, src_ui, dst_ui, agg_i, deg_i)


def _messages(h_user, h_item, src_iu, dst_iu, src_ui, dst_ui):
    mesh = plsc.VectorSubcoreMesh(core_axis_name="c", subcore_axis_name="s")
    z80 = jnp.zeros((KU, D), jnp.float32)
    ones = jnp.ones((KU, D), jnp.float32)
    row_iota = jnp.arange(NP, dtype=jnp.int32).reshape(
        NS, ROWS_PER_TILE // KU, KU)
    kern = functools.partial(
        pl.kernel,
        out_type=(
            jax.ShapeDtypeStruct((NP, D), jnp.float32),
            jax.ShapeDtypeStruct((NP, D), jnp.float32),
            jax.ShapeDtypeStruct((NP, D), jnp.float32),
            jax.ShapeDtypeStruct((NP, D), jnp.float32),
        ),
        mesh=mesh,
        scratch_types=[
            pltpu.VMEM((UNITS_PER_BLOCK, KU), jnp.int32),
            pltpu.VMEM((UNITS_PER_BLOCK, KU), jnp.int32),
            pltpu.VMEM((KU, D), jnp.float32),
            pltpu.VMEM((KU, D), jnp.float32),
            pltpu.VMEM((ROWS_PER_TILE // KU, KU), jnp.int32),
            pltpu.VMEM((KU,), jnp.int32),
            pltpu.VMEM((KU,), jnp.int32),
            pltpu.VMEM((KU,), jnp.int32),
            pltpu.SemaphoreType.DMA,
            pltpu.SemaphoreType.DMA,
            pltpu.SemaphoreType.DMA,
            pltpu.VMEM_SHARED((NP, D), jnp.float32),
        ],
    )(_sc_body)
    return kern(h_item, h_user, src_iu, dst_iu, src_ui, dst_ui,
                z80, ones, row_iota)


def _epi_body(scal_ref, agg_u, deg_u, h_u, agg_i, deg_i, h_i, wt, bo,
              out_u, out_i):
    al_iu = scal_ref[0, 0]
    ar_iu = scal_ref[0, 1]
    al_ui = scal_ref[0, 2]
    ar_ui = scal_ref[0, 3]
    pre_u = agg_u[...] * al_iu + h_u[...] * (deg_u[:, 0:1] * ar_iu)
    pre_u = jnp.where(pre_u > 0, pre_u, jnp.exp(pre_u) - 1.0)
    out_u[...] = (
        jnp.dot(pre_u, wt[...], preferred_element_type=jnp.float32) + bo[...]
    )
    pre_i = agg_i[...] * al_ui + h_i[...] * (deg_i[:, 0:1] * ar_ui)
    out_i[...] = jnp.where(pre_i > 0, pre_i, jnp.exp(pre_i) - 1.0)


def _epilogue(scal, agg_u, deg_u, h_u, agg_i, deg_i, h_i, wt, bo2):
    bn = 2000
    return pl.pallas_call(
        _epi_body,
        grid=(N // bn,),
        in_specs=[
            pl.BlockSpec(memory_space=pltpu.SMEM),
            pl.BlockSpec((bn, D), lambda i: (i, 0)),
            pl.BlockSpec((bn, D), lambda i: (i, 0)),
            pl.BlockSpec((bn, D), lambda i: (i, 0)),
            pl.BlockSpec((bn, D), lambda i: (i, 0)),
            pl.BlockSpec((bn, D), lambda i: (i, 0)),
            pl.BlockSpec((bn, D), lambda i: (i, 0)),
            pl.BlockSpec((D, D), lambda i: (0, 0)),
            pl.BlockSpec((1, D), lambda i: (0, 0)),
        ],
        out_specs=[
            pl.BlockSpec((bn, D), lambda i: (i, 0)),
            pl.BlockSpec((bn, D), lambda i: (i, 0)),
        ],
        out_shape=[
            jax.ShapeDtypeStruct((N, D), jnp.float32),
            jax.ShapeDtypeStruct((N, D), jnp.float32),
        ],
    )(scal, agg_u, deg_u, h_u, agg_i, deg_i, h_i, wt, bo2)


def kernel(x_user, x_item, edge_item_user, edge_user_item,
           W_proj, b_proj, W_out, b_out,
           attn_l_iu, attn_r_iu, attn_l_ui, attn_r_ui):
    wpt = W_proj.T
    wot = W_out.T
    bp2 = b_proj.reshape(1, D)
    bo2 = b_out.reshape(1, D)

    h_user = _project(x_user, wpt, bp2)
    h_item = _project(x_item, wpt, bp2)

    def _pack(edge):
        e = edge.astype(jnp.int32)
        shp = (NS, IDX_BLOCKS, UNITS_PER_BLOCK, KU)
        return e[0].reshape(shp), e[1].reshape(shp)

    src_iu, dst_iu = _pack(edge_item_user)
    src_ui, dst_ui = _pack(edge_user_item)

    agg_u, deg_u, agg_i, deg_i = _messages(
        h_user, h_item, src_iu, dst_iu, src_ui, dst_ui)

    scal = jnp.stack(
        [attn_l_iu[0], attn_r_iu[0], attn_l_ui[0], attn_r_ui[0]]
    ).reshape(1, 4)
    out_user, out_item = _epilogue(
        scal, agg_u, deg_u, h_user, agg_i, deg_i, h_item, wot, bo2)
    return out_user, out_item
